# baseline (speedup 1.0000x reference)
import functools

import jax
import jax.numpy as jnp
from jax import lax
from jax.experimental import pallas as pl
from jax.experimental.pallas import tpu as pltpu
from jax.experimental.pallas import tpu_sc as plsc

EMB_DIM = 32
NEG_K = 20
NUM_CORES = 2
NUM_SUBCORES = 16
NUM_WORKERS = NUM_CORES * NUM_SUBCORES
CHUNK = 128
GROUP = 16
GATHER = 128
RBLK = 128


def _sc_linearize(t0, t1):
    v = t0.shape[2]
    n_blk = v // RBLK
    tail = v - n_blk * RBLK
    per_w = n_blk // NUM_WORKERS
    extra = n_blk - per_w * NUM_WORKERS
    mesh = plsc.VectorSubcoreMesh(core_axis_name="c", subcore_axis_name="s")
    out_t = jax.ShapeDtypeStruct((v // 4, 128), jnp.float32)

    @functools.partial(
        pl.kernel,
        out_type=(out_t, out_t),
        mesh=mesh,
        scratch_types=[
            pltpu.VMEM((EMB_DIM, RBLK), jnp.float32),
            pltpu.VMEM((EMB_DIM, RBLK), jnp.float32),
            pltpu.VMEM((RBLK // 4, 128), jnp.float32),
            pltpu.VMEM((RBLK // 4, 128), jnp.float32),
            pltpu.VMEM((EMB_DIM, 64), jnp.float32),
            pltpu.VMEM((EMB_DIM, 64), jnp.float32),
            pltpu.SemaphoreType.DMA,
        ],
        compiler_params=pltpu.CompilerParams(needs_layout_passes=False),
    )
    def conv_kernel(t0_hbm, t1_hbm, o0_hbm, o1_hbm, v0, v1, ob0, ob1,
                    vt0, vt1, sem):
        wid = lax.axis_index("s") * NUM_CORES + lax.axis_index("c")
        iota = lax.iota(jnp.int32, GROUP)
        base = wid * per_w + jnp.minimum(wid, extra)
        cnt = jnp.where(wid < extra, per_w + 1, per_w)

        def transpose_block(vbuf, obuf, ncols, s_base=0):
            for s in range(ncols // 4):
                for jj in range(8):
                    rows = iota + (jj % 2) * GROUP
                    col = jnp.full((GROUP,), 4 * s + jj // 2, jnp.int32)
                    g = plsc.load_gather(vbuf, [rows, col])
                    obuf[s_base + s, pl.ds(jj * GROUP, GROUP)] = g

        def blk_body(i, carry):
            r0 = pl.multiple_of((base + i) * RBLK, RBLK)
            copies = []
            for b in range(4):
                copies.append(pltpu.async_copy(
                    t0_hbm.at[b, :, pl.ds(r0, RBLK)],
                    v0.at[pl.ds(b * 8, 8)], sem))
                copies.append(pltpu.async_copy(
                    t1_hbm.at[b, :, pl.ds(r0, RBLK)],
                    v1.at[pl.ds(b * 8, 8)], sem))
            for cp in copies:
                cp.wait()
            transpose_block(v0, ob0, RBLK)
            transpose_block(v1, ob1, RBLK)
            s0 = pl.multiple_of((base + i) * (RBLK // 4), RBLK // 4)
            pltpu.sync_copy(ob0, o0_hbm.at[pl.ds(s0, RBLK // 4)])
            pltpu.sync_copy(ob1, o1_hbm.at[pl.ds(s0, RBLK // 4)])
            return carry

        lax.fori_loop(0, cnt, blk_body, 0)

        if tail:
            @pl.when(wid == NUM_WORKERS - 1)
            def _():
                r0 = n_blk * RBLK
                copies = []
                for b in range(4):
                    for dd in range(8):
                        copies.append(pltpu.async_copy(
                            t0_hbm.at[b, dd, pl.ds(r0, tail)],
                            vt0.at[b * 8 + dd], sem))
                        copies.append(pltpu.async_copy(
                            t1_hbm.at[b, dd, pl.ds(r0, tail)],
                            vt1.at[b * 8 + dd], sem))
                for cp in copies:
                    cp.wait()
                transpose_block(vt0, ob0, tail)
                transpose_block(vt1, ob1, tail)
                s0 = r0 // 4
                pltpu.sync_copy(ob0.at[pl.ds(0, tail // 4)],
                                o0_hbm.at[pl.ds(s0, tail // 4)])
                pltpu.sync_copy(ob1.at[pl.ds(0, tail // 4)],
                                o1_hbm.at[pl.ds(s0, tail // 4)])

    return conv_kernel(t0, t1)


def _sc_dots(cen_idx, ctx_idx, neg_idx, syn0, syn1):
    B = cen_idx.shape[0]
    per_w = B // NUM_WORKERS
    n_chunks = per_w // CHUNK
    out_per_chunk = CHUNK * (1 + NEG_K)
    mesh = plsc.VectorSubcoreMesh(core_axis_name="c", subcore_axis_name="s")

    @functools.partial(
        pl.kernel,
        out_type=jax.ShapeDtypeStruct((B * (1 + NEG_K),), jnp.float32),
        mesh=mesh,
        scratch_types=[
            pltpu.VMEM((CHUNK,), jnp.int32),
            pltpu.VMEM((CHUNK,), jnp.int32),
            pltpu.VMEM((CHUNK * NEG_K,), jnp.int32),
            pltpu.VMEM((CHUNK, EMB_DIM), jnp.float32),
            pltpu.VMEM((CHUNK, EMB_DIM), jnp.float32),
            pltpu.VMEM((CHUNK * NEG_K, EMB_DIM), jnp.float32),
            pltpu.VMEM((CHUNK * (1 + NEG_K),), jnp.float32),
            pltpu.SemaphoreType.DMA,
        ],
        compiler_params=pltpu.CompilerParams(
            needs_layout_passes=False, use_tc_tiling_on_sc=False),
    )
    def sc_kernel(cen_hbm, ctx_hbm, neg_hbm, syn0_hbm, syn1_hbm, out_hbm,
                  cen_i, ctx_i, neg_i, cen_r, ctx_r, neg_r, ob, sem):
        wid = lax.axis_index("s") * NUM_CORES + lax.axis_index("c")
        iota = lax.iota(jnp.int32, GROUP)
        cols = [jnp.full((GROUP,), d, jnp.int32) for d in range(EMB_DIM)]

        def chunk_body(c, carry):
            base = wid * per_w + c * CHUNK
            pltpu.sync_copy(cen_hbm.at[pl.ds(base, CHUNK)], cen_i)
            pltpu.sync_copy(ctx_hbm.at[pl.ds(base, CHUNK)], ctx_i)
            pltpu.sync_copy(neg_hbm.at[pl.ds(base * NEG_K, CHUNK * NEG_K)], neg_i)
            copies = [
                pltpu.async_copy(syn0_hbm.at[cen_i], cen_r, sem),
                pltpu.async_copy(syn1_hbm.at[ctx_i], ctx_r, sem),
            ]
            for j in range(CHUNK * NEG_K // GATHER):
                copies.append(pltpu.async_copy(
                    syn1_hbm.at[neg_i.at[pl.ds(j * GATHER, GATHER)]],
                    neg_r.at[pl.ds(j * GATHER, GATHER)], sem))
            for cp in copies:
                cp.wait()

            def group_body(g, gcarry):
                e = g * GROUP + iota
                cen_d = [plsc.load_gather(cen_r, [e, cols[d]])
                         for d in range(EMB_DIM)]

                def dot_against(rows_ref, row_idx):
                    p = [cen_d[d] * plsc.load_gather(rows_ref, [row_idx, cols[d]])
                         for d in range(EMB_DIM)]
                    while len(p) > 1:
                        p = [p[i] + p[i + 1] for i in range(0, len(p), 2)]
                    return p[0]

                ob[pl.ds(g * GROUP, GROUP)] = dot_against(ctx_r, e)
                e_neg = e * NEG_K
                unroll = 4

                def neg_body(kq, kcarry):
                    kk0 = kq * unroll
                    for u in range(unroll):
                        acc = dot_against(neg_r, e_neg + (kk0 + u))
                        ob[pl.ds(CHUNK + (kk0 + u) * CHUNK + g * GROUP,
                                 GROUP)] = -acc
                    return kcarry

                lax.fori_loop(0, NEG_K // unroll, neg_body, 0)
                return gcarry

            lax.fori_loop(0, CHUNK // GROUP, group_body, 0)
            pltpu.sync_copy(
                ob,
                out_hbm.at[pl.ds((wid * n_chunks + c) * out_per_chunk,
                                 out_per_chunk)])
            return carry

        lax.fori_loop(0, n_chunks, chunk_body, 0)

    return sc_kernel(cen_idx, ctx_idx, neg_idx, syn0, syn1)


def _tc_loss(dots):
    n = dots.shape[0]
    x2 = dots.reshape(n // 128, 128)

    def body(x_ref, o_ref):
        x = x_ref[...]
        ls = jnp.minimum(x, 0.0) - jnp.log1p(jnp.exp(-jnp.abs(x)))
        o_ref[0, 0] = -jnp.sum(jnp.sum(ls, axis=1))

    out = pl.pallas_call(
        body,
        out_shape=jax.ShapeDtypeStruct((1, 1), jnp.float32),
        out_specs=pl.BlockSpec(memory_space=pltpu.SMEM),
    )(x2)
    return out[0, 0]


def kernel(center_word, context_word, neg_sampling_words, syn0, syn1):
    cen = center_word.astype(jnp.int32)
    ctx = context_word.astype(jnp.int32)
    neg = neg_sampling_words.astype(jnp.int32).reshape(-1)
    v = syn0.shape[0]
    t0 = syn0.T.reshape(4, 8, v)
    t1 = syn1.T.reshape(4, 8, v)
    lin0, lin1 = _sc_linearize(t0, t1)
    dots = _sc_dots(cen, ctx, neg,
                    lin0.reshape(v, EMB_DIM), lin1.reshape(v, EMB_DIM))
    return _tc_loss(dots)


# speedup vs baseline: 1.2730x; 1.2730x over previous
import functools

import jax
import jax.numpy as jnp
from jax import lax
from jax.experimental import pallas as pl
from jax.experimental.pallas import tpu as pltpu
from jax.experimental.pallas import tpu_sc as plsc

EMB_DIM = 32
NEG_K = 20
NUM_CORES = 2
NUM_SUBCORES = 16
NUM_WORKERS = NUM_CORES * NUM_SUBCORES
CHUNK = 128
GROUP = 16
GATHER = 128
RBLK = 512


def _sc_linearize(t0, t1):
    v = t0.shape[2]
    n_blk = v // RBLK
    tail = v - n_blk * RBLK
    per_w = n_blk // NUM_WORKERS
    extra = n_blk - per_w * NUM_WORKERS
    mesh = plsc.VectorSubcoreMesh(core_axis_name="c", subcore_axis_name="s")
    out_t = jax.ShapeDtypeStruct((v // 4, 128), jnp.float32)

    @functools.partial(
        pl.kernel,
        out_type=(out_t, out_t),
        mesh=mesh,
        scratch_types=[
            pltpu.VMEM((EMB_DIM, RBLK), jnp.float32),
            pltpu.VMEM((EMB_DIM, RBLK), jnp.float32),
            pltpu.VMEM((EMB_DIM, RBLK), jnp.float32),
            pltpu.VMEM((EMB_DIM, RBLK), jnp.float32),
            pltpu.VMEM((RBLK // 4, 128), jnp.float32),
            pltpu.VMEM((RBLK // 4, 128), jnp.float32),
            pltpu.VMEM((EMB_DIM, 64), jnp.float32),
            pltpu.VMEM((EMB_DIM, 64), jnp.float32),
            pltpu.SemaphoreType.DMA,
            pltpu.SemaphoreType.DMA,
        ],
        compiler_params=pltpu.CompilerParams(needs_layout_passes=False),
    )
    def conv_kernel(t0_hbm, t1_hbm, o0_hbm, o1_hbm, v0a, v0b, v1a, v1b,
                    ob0, ob1, vt0, vt1, sem_in, sem_out):
        wid = lax.axis_index("s") * NUM_CORES + lax.axis_index("c")
        iota = lax.iota(jnp.int32, GROUP)
        base = wid * per_w + jnp.minimum(wid, extra)
        cnt = jnp.where(wid < extra, per_w + 1, per_w)
        vbufs = ((v0a, v1a), (v0b, v1b))

        def in_descs(i, par):
            r0 = pl.multiple_of((base + i) * RBLK, RBLK)
            vb0, vb1 = vbufs[par]
            ds_ = []
            for b in range(4):
                ds_.append(pltpu.make_async_copy(
                    t0_hbm.at[b, :, pl.ds(r0, RBLK)],
                    vb0.at[pl.ds(b * 8, 8)], sem_in))
                ds_.append(pltpu.make_async_copy(
                    t1_hbm.at[b, :, pl.ds(r0, RBLK)],
                    vb1.at[pl.ds(b * 8, 8)], sem_in))
            return ds_

        def out_descs(i):
            s0 = pl.multiple_of((base + i) * (RBLK // 4), RBLK // 4)
            return [
                pltpu.make_async_copy(ob0, o0_hbm.at[pl.ds(s0, RBLK // 4)],
                                      sem_out),
                pltpu.make_async_copy(ob1, o1_hbm.at[pl.ds(s0, RBLK // 4)],
                                      sem_out),
            ]

        def transpose_block(vbuf, obuf, ncols, unroll_cols=True):
            def one_s(s):
                for jj in range(8):
                    rows = iota + (jj % 2) * GROUP
                    col = jnp.full((GROUP,), 4 * s + jj // 2, jnp.int32)
                    g = plsc.load_gather(vbuf, [rows, col])
                    obuf[s, pl.ds(jj * GROUP, GROUP)] = g

            if unroll_cols:
                def s_body(ss, carry):
                    for s8 in range(8):
                        one_s(ss * 8 + s8)
                    return carry
                lax.fori_loop(0, ncols // 4 // 8, s_body, 0)
            else:
                for s in range(ncols // 4):
                    one_s(s)

        for d in in_descs(0, 0):
            d.start()

        def pair_body(p, carry):
            for par in range(2):
                i = 2 * p + par

                @pl.when(i < cnt)
                def _():
                    for d in in_descs(i, par):
                        d.wait()

                    @pl.when(i + 1 < cnt)
                    def _():
                        for d in in_descs(i + 1, 1 - par):
                            d.start()

                    @pl.when(i > 0)
                    def _():
                        for d in out_descs(i - 1):
                            d.wait()

                    vb0, vb1 = vbufs[par]
                    transpose_block(vb0, ob0, RBLK)
                    transpose_block(vb1, ob1, RBLK)
                    for d in out_descs(i):
                        d.start()
            return carry

        lax.fori_loop(0, (cnt + 1) // 2, pair_body, 0)
        for d in out_descs(cnt - 1):
            d.wait()

        if tail:
            @pl.when(wid == NUM_WORKERS - 1)
            def _():
                r0 = n_blk * RBLK
                copies = []
                for b in range(4):
                    for dd in range(8):
                        copies.append(pltpu.async_copy(
                            t0_hbm.at[b, dd, pl.ds(r0, tail)],
                            vt0.at[b * 8 + dd], sem_in))
                        copies.append(pltpu.async_copy(
                            t1_hbm.at[b, dd, pl.ds(r0, tail)],
                            vt1.at[b * 8 + dd], sem_in))
                for cp in copies:
                    cp.wait()
                transpose_block(vt0, ob0, tail)
                transpose_block(vt1, ob1, tail)
                s0 = r0 // 4
                pltpu.sync_copy(ob0.at[pl.ds(0, tail // 4)],
                                o0_hbm.at[pl.ds(s0, tail // 4)])
                pltpu.sync_copy(ob1.at[pl.ds(0, tail // 4)],
                                o1_hbm.at[pl.ds(s0, tail // 4)])

    return conv_kernel(t0, t1)


def _sc_dots(cen_idx, ctx_idx, neg_idx, syn0, syn1):
    B = cen_idx.shape[0]
    per_w = B // NUM_WORKERS
    n_chunks = per_w // CHUNK
    out_per_chunk = CHUNK * (1 + NEG_K)
    mesh = plsc.VectorSubcoreMesh(core_axis_name="c", subcore_axis_name="s")

    @functools.partial(
        pl.kernel,
        out_type=jax.ShapeDtypeStruct((B * (1 + NEG_K),), jnp.float32),
        mesh=mesh,
        scratch_types=[
            pltpu.VMEM((CHUNK,), jnp.int32),
            pltpu.VMEM((CHUNK,), jnp.int32),
            pltpu.VMEM((CHUNK * NEG_K,), jnp.int32),
            pltpu.VMEM((CHUNK, EMB_DIM), jnp.float32),
            pltpu.VMEM((CHUNK, EMB_DIM), jnp.float32),
            pltpu.VMEM((CHUNK * NEG_K, EMB_DIM), jnp.float32),
            pltpu.VMEM((CHUNK * (1 + NEG_K),), jnp.float32),
            pltpu.SemaphoreType.DMA,
        ],
        compiler_params=pltpu.CompilerParams(
            needs_layout_passes=False, use_tc_tiling_on_sc=False),
    )
    def sc_kernel(cen_hbm, ctx_hbm, neg_hbm, syn0_hbm, syn1_hbm, out_hbm,
                  cen_i, ctx_i, neg_i, cen_r, ctx_r, neg_r, ob, sem):
        wid = lax.axis_index("s") * NUM_CORES + lax.axis_index("c")
        iota = lax.iota(jnp.int32, GROUP)
        cols = [jnp.full((GROUP,), d, jnp.int32) for d in range(EMB_DIM)]

        def chunk_body(c, carry):
            base = wid * per_w + c * CHUNK
            pltpu.sync_copy(cen_hbm.at[pl.ds(base, CHUNK)], cen_i)
            pltpu.sync_copy(ctx_hbm.at[pl.ds(base, CHUNK)], ctx_i)
            pltpu.sync_copy(neg_hbm.at[pl.ds(base * NEG_K, CHUNK * NEG_K)], neg_i)
            copies = [
                pltpu.async_copy(syn0_hbm.at[cen_i], cen_r, sem),
                pltpu.async_copy(syn1_hbm.at[ctx_i], ctx_r, sem),
            ]
            for j in range(CHUNK * NEG_K // GATHER):
                copies.append(pltpu.async_copy(
                    syn1_hbm.at[neg_i.at[pl.ds(j * GATHER, GATHER)]],
                    neg_r.at[pl.ds(j * GATHER, GATHER)], sem))
            for cp in copies:
                cp.wait()

            def group_body(g, gcarry):
                e = g * GROUP + iota
                cen_d = [plsc.load_gather(cen_r, [e, cols[d]])
                         for d in range(EMB_DIM)]

                def dot_against(rows_ref, row_idx):
                    p = [cen_d[d] * plsc.load_gather(rows_ref, [row_idx, cols[d]])
                         for d in range(EMB_DIM)]
                    while len(p) > 1:
                        p = [p[i] + p[i + 1] for i in range(0, len(p), 2)]
                    return p[0]

                ob[pl.ds(g * GROUP, GROUP)] = dot_against(ctx_r, e)
                e_neg = e * NEG_K
                unroll = 4

                def neg_body(kq, kcarry):
                    kk0 = kq * unroll
                    for u in range(unroll):
                        acc = dot_against(neg_r, e_neg + (kk0 + u))
                        ob[pl.ds(CHUNK + (kk0 + u) * CHUNK + g * GROUP,
                                 GROUP)] = -acc
                    return kcarry

                lax.fori_loop(0, NEG_K // unroll, neg_body, 0)
                return gcarry

            lax.fori_loop(0, CHUNK // GROUP, group_body, 0)
            pltpu.sync_copy(
                ob,
                out_hbm.at[pl.ds((wid * n_chunks + c) * out_per_chunk,
                                 out_per_chunk)])
            return carry

        lax.fori_loop(0, n_chunks, chunk_body, 0)

    return sc_kernel(cen_idx, ctx_idx, neg_idx, syn0, syn1)


def _tc_loss(dots):
    n = dots.shape[0]
    x2 = dots.reshape(n // 128, 128)

    def body(x_ref, o_ref):
        x = x_ref[...]
        ls = jnp.minimum(x, 0.0) - jnp.log1p(jnp.exp(-jnp.abs(x)))
        o_ref[0, 0] = -jnp.sum(jnp.sum(ls, axis=1))

    out = pl.pallas_call(
        body,
        out_shape=jax.ShapeDtypeStruct((1, 1), jnp.float32),
        out_specs=pl.BlockSpec(memory_space=pltpu.SMEM),
    )(x2)
    return out[0, 0]


def kernel(center_word, context_word, neg_sampling_words, syn0, syn1):
    cen = center_word.astype(jnp.int32)
    ctx = context_word.astype(jnp.int32)
    neg = neg_sampling_words.astype(jnp.int32).reshape(-1)
    v = syn0.shape[0]
    t0 = syn0.T.reshape(4, 8, v)
    t1 = syn1.T.reshape(4, 8, v)
    lin0, lin1 = _sc_linearize(t0, t1)
    dots = _sc_dots(cen, ctx, neg,
                    lin0.reshape(v, EMB_DIM), lin1.reshape(v, EMB_DIM))
    return _tc_loss(dots)


# speedup vs baseline: 2.1569x; 1.6943x over previous
import functools

import jax
import jax.numpy as jnp
from jax import lax
from jax.experimental import pallas as pl
from jax.experimental.pallas import tpu as pltpu
from jax.experimental.pallas import tpu_sc as plsc

EMB_DIM = 32
NEG_K = 20
NUM_CORES = 2
NUM_SUBCORES = 16
NUM_WORKERS = NUM_CORES * NUM_SUBCORES
CHUNK = 128
GROUP = 16
GATHER = 128


def _sc_dots(cen_idx, ctx_idx, neg_idx, syn0, syn1):
    B = cen_idx.shape[0]
    per_w = B // NUM_WORKERS
    n_chunks = per_w // CHUNK
    out_per_chunk = CHUNK * (1 + NEG_K)
    mesh = plsc.VectorSubcoreMesh(core_axis_name="c", subcore_axis_name="s")

    @functools.partial(
        pl.kernel,
        out_type=jax.ShapeDtypeStruct((B * (1 + NEG_K),), jnp.float32),
        mesh=mesh,
        scratch_types=[
            pltpu.VMEM((CHUNK,), jnp.int32),
            pltpu.VMEM((CHUNK,), jnp.int32),
            pltpu.VMEM((CHUNK * NEG_K,), jnp.int32),
            pltpu.VMEM((CHUNK, EMB_DIM), jnp.float32),
            pltpu.VMEM((CHUNK, EMB_DIM), jnp.float32),
            pltpu.VMEM((CHUNK * NEG_K, EMB_DIM), jnp.float32),
            pltpu.VMEM((CHUNK * (1 + NEG_K),), jnp.float32),
            pltpu.SemaphoreType.DMA,
        ],
        compiler_params=pltpu.CompilerParams(
            needs_layout_passes=False, use_tc_tiling_on_sc=False),
    )
    def sc_kernel(cen_hbm, ctx_hbm, neg_hbm, syn0_hbm, syn1_hbm, out_hbm,
                  cen_i, ctx_i, neg_i, cen_r, ctx_r, neg_r, ob, sem):
        wid = lax.axis_index("s") * NUM_CORES + lax.axis_index("c")
        iota = lax.iota(jnp.int32, GROUP)
        cols = [jnp.full((GROUP,), d, jnp.int32) for d in range(EMB_DIM)]

        def chunk_body(c, carry):
            base = wid * per_w + c * CHUNK
            pltpu.sync_copy(cen_hbm.at[pl.ds(base, CHUNK)], cen_i)
            pltpu.sync_copy(ctx_hbm.at[pl.ds(base, CHUNK)], ctx_i)
            pltpu.sync_copy(neg_hbm.at[pl.ds(base * NEG_K, CHUNK * NEG_K)], neg_i)
            copies = [
                pltpu.async_copy(syn0_hbm.at[cen_i], cen_r, sem),
                pltpu.async_copy(syn1_hbm.at[ctx_i], ctx_r, sem),
            ]
            for j in range(CHUNK * NEG_K // GATHER):
                copies.append(pltpu.async_copy(
                    syn1_hbm.at[neg_i.at[pl.ds(j * GATHER, GATHER)]],
                    neg_r.at[pl.ds(j * GATHER, GATHER)], sem))
            for cp in copies:
                cp.wait()

            def group_body(g, gcarry):
                e = g * GROUP + iota
                cen_d = [plsc.load_gather(cen_r, [e, cols[d]])
                         for d in range(EMB_DIM)]

                def dot_against(rows_ref, row_idx):
                    p = [cen_d[d] * plsc.load_gather(rows_ref, [row_idx, cols[d]])
                         for d in range(EMB_DIM)]
                    while len(p) > 1:
                        p = [p[i] + p[i + 1] for i in range(0, len(p), 2)]
                    return p[0]

                ob[pl.ds(g * GROUP, GROUP)] = dot_against(ctx_r, e)
                e_neg = e * NEG_K
                unroll = 4

                def neg_body(kq, kcarry):
                    kk0 = kq * unroll
                    for u in range(unroll):
                        acc = dot_against(neg_r, e_neg + (kk0 + u))
                        ob[pl.ds(CHUNK + (kk0 + u) * CHUNK + g * GROUP,
                                 GROUP)] = -acc
                    return kcarry

                lax.fori_loop(0, NEG_K // unroll, neg_body, 0)
                return gcarry

            lax.fori_loop(0, CHUNK // GROUP, group_body, 0)
            pltpu.sync_copy(
                ob,
                out_hbm.at[pl.ds((wid * n_chunks + c) * out_per_chunk,
                                 out_per_chunk)])
            return carry

        lax.fori_loop(0, n_chunks, chunk_body, 0)

    return sc_kernel(cen_idx, ctx_idx, neg_idx, syn0, syn1)


def _tc_loss(dots):
    n = dots.shape[0]
    x2 = dots.reshape(n // 128, 128)

    def body(x_ref, o_ref):
        x = x_ref[...]
        ls = jnp.minimum(x, 0.0) - jnp.log1p(jnp.exp(-jnp.abs(x)))
        o_ref[0, 0] = -jnp.sum(jnp.sum(ls, axis=1))

    out = pl.pallas_call(
        body,
        out_shape=jax.ShapeDtypeStruct((1, 1), jnp.float32),
        out_specs=pl.BlockSpec(memory_space=pltpu.SMEM),
    )(x2)
    return out[0, 0]


def kernel(center_word, context_word, neg_sampling_words, syn0, syn1):
    cen = center_word.astype(jnp.int32)
    ctx = context_word.astype(jnp.int32)
    neg = neg_sampling_words.astype(jnp.int32).reshape(-1)
    dots = _sc_dots(cen, ctx, neg, syn0, syn1)
    return _tc_loss(dots)
